# baseline (speedup 1.0000x reference)
import functools

import jax
import jax.numpy as jnp
import numpy as np
from jax.experimental import pallas as pl
from jax.experimental.pallas import tpu as pltpu
from jax.sharding import Mesh, PartitionSpec as P

_EPS = 1e-5
_LANES = 128
_VMEM = 48 * 1024 * 1024


def _expansion_matrix(G, W):
    p = np.arange(G)
    base = (4 * (p // W) * W + 2 * (p % W))[:, None]
    q = np.arange(4 * G)[None, :]
    hit = (q == base) | (q == base + 1) | (q == base + 2 * W) | (q == base + 2 * W + 1)
    return hit.astype(np.float32)


def _forward(x, w, gamma, beta, n_global, axis):
    N, C, H, W = x.shape
    Cout = w.shape[1]
    HW = H * W
    inv_cnt = 1.0 / float(n_global * HW)

    G = W
    while G < 4 * W and G < 4 * _LANES and HW % (2 * G) == 0:
        G *= 2
    n_groups = HW // G
    slab_rows = 2 * G // W
    lane_chunks = HW // _LANES if HW % _LANES == 0 else 0

    x3 = x.reshape(N, C, HW)

    def stats_kernel(x_ref, s_ref, q_ref):
        xv = x_ref[...]
        if lane_chunks:
            s = xv[:, 0:_LANES]
            q = s * s
            for k in range(1, lane_chunks):
                c = xv[:, k * _LANES:(k + 1) * _LANES]
                s = s + c
                q = q + c * c
            s_ref[...] = jnp.sum(s, axis=-1, keepdims=True)
            q_ref[...] = jnp.sum(q, axis=-1, keepdims=True)
        else:
            s_ref[...] = jnp.sum(xv, axis=-1, keepdims=True)
            q_ref[...] = jnp.sum(xv * xv, axis=-1, keepdims=True)

    sum_c, sq_c = pl.pallas_call(
        stats_kernel,
        out_shape=(jax.ShapeDtypeStruct((N, C, 1), jnp.float32),
                   jax.ShapeDtypeStruct((N, C, 1), jnp.float32)),
        grid=(N,),
        in_specs=[pl.BlockSpec((None, C, HW), lambda n: (n, 0, 0))],
        out_specs=(pl.BlockSpec((None, C, 1), lambda n: (n, 0, 0)),
                   pl.BlockSpec((None, C, 1), lambda n: (n, 0, 0))),
        compiler_params=pltpu.CompilerParams(
            dimension_semantics=("arbitrary",),
            vmem_limit_bytes=_VMEM),
    )(x3)

    s1 = jnp.sum(sum_c, axis=0)
    s2 = jnp.sum(sq_c, axis=0)
    if axis is not None:
        s1 = jax.lax.psum(s1, axis)
        s2 = jax.lax.psum(s2, axis)
    mean = s1 * inv_cnt
    var = jnp.maximum(s2 * inv_cnt - mean * mean, 0.0)
    scale = gamma.reshape(C, 1) * jax.lax.rsqrt(var + _EPS)
    shift = beta.reshape(C, 1) - mean * scale

    wt16 = w.T.astype(jnp.bfloat16)
    d16 = jnp.asarray(_expansion_matrix(G, W), dtype=jnp.bfloat16)

    def fused_kernel(sc_ref, sh_ref, wt_ref, d_ref, x_ref, o_ref):
        xv = x_ref[...]
        xr = jnp.maximum(xv * sc_ref[...] + sh_ref[...], 0.0)
        y16 = jnp.dot(wt_ref[...], xr.astype(jnp.bfloat16),
                      preferred_element_type=jnp.float32
                      ).astype(jnp.bfloat16)
        d = d_ref[...]
        for gi in range(n_groups):
            z = jnp.dot(y16[:, gi * G:(gi + 1) * G], d,
                        preferred_element_type=jnp.float32)
            slab = z.reshape(Cout, slab_rows, 2 * W)
            o_ref[:, gi * slab_rows:(gi + 1) * slab_rows, :] = slab

    out = pl.pallas_call(
        fused_kernel,
        out_shape=jax.ShapeDtypeStruct((N, Cout, 2 * H, 2 * W), jnp.float32),
        grid=(N,),
        in_specs=[
            pl.BlockSpec((C, 1), lambda n: (0, 0)),
            pl.BlockSpec((C, 1), lambda n: (0, 0)),
            pl.BlockSpec((Cout, C), lambda n: (0, 0)),
            pl.BlockSpec((G, 4 * G), lambda n: (0, 0)),
            pl.BlockSpec((None, C, HW), lambda n: (n, 0, 0)),
        ],
        out_specs=pl.BlockSpec((None, Cout, 2 * H, 2 * W),
                               lambda n: (n, 0, 0, 0)),
        compiler_params=pltpu.CompilerParams(
            dimension_semantics=("arbitrary",),
            vmem_limit_bytes=_VMEM),
    )(scale, shift, wt16, d16, x3)

    return out


def kernel(x, w, gamma, beta):
    N = x.shape[0]
    devices = jax.devices()
    n_dev = 2 if len(devices) >= 2 and N % 2 == 0 else 1
    if n_dev == 1:
        return _forward(x, w, gamma, beta, n_global=N, axis=None)
    mesh = Mesh(np.array(devices[:2]), ("b",))
    fn = jax.shard_map(
        functools.partial(_forward, n_global=N, axis="b"),
        mesh=mesh,
        in_specs=(P("b"), P(), P(), P()),
        out_specs=P("b"),
        check_vma=False,
    )
    return fn(x, w, gamma, beta)


# speedup vs baseline: 4.8116x; 4.8116x over previous
import jax
import jax.numpy as jnp
import numpy as np
from jax.experimental import pallas as pl
from jax.experimental.pallas import tpu as pltpu

_EPS = 1e-5
_LANES = 128
_VMEM = 56 * 1024 * 1024


def _width_double_matrix(G, W):
    p = np.arange(G)
    base = (4 * (p // W) * W + 2 * (p % W))[:, None]
    q = np.arange(4 * G)[None, :]
    hit = (q == base) | (q == base + 1) | (q == base + 2 * W) | (q == base + 2 * W + 1)
    return hit.astype(np.float32)


def kernel(x, w, gamma, beta):
    N, C, H, W = x.shape
    Cout = w.shape[1]
    HW = H * W
    inv_cnt = 1.0 / float(N * HW)

    G = W
    while G < 4 * W and G < 4 * _LANES and HW % (2 * G) == 0:
        G *= 2
    rows_per_g = G // W
    n_groups = HW // G
    lane_chunks = HW // _LANES if HW % _LANES == 0 else 0

    x3 = x.reshape(N, C, HW)
    gamma2 = gamma.reshape(C, 1).astype(jnp.float32)
    beta2 = beta.reshape(C, 1).astype(jnp.float32)
    wt16 = w.T.astype(jnp.bfloat16)
    d16 = jnp.asarray(_width_double_matrix(G, W), dtype=jnp.bfloat16)

    def body(g_ref, b_ref, wt_ref, d_ref, x_ref, o_ref,
             xbuf, acc_s, acc_q, scale_r, shift_r):
        i = pl.program_id(0)

        @pl.when(i < N)
        def _phase_a():
            xv = x_ref[...]

            @pl.when(i == 0)
            def _init():
                acc_s[...] = jnp.zeros_like(acc_s)
                acc_q[...] = jnp.zeros_like(acc_q)

            if lane_chunks:
                s = xv[:, 0:_LANES]
                q = s * s
                for k in range(1, lane_chunks):
                    c = xv[:, k * _LANES:(k + 1) * _LANES]
                    s = s + c
                    q = q + c * c
                acc_s[...] += s
                acc_q[...] += q
            else:
                acc_s[:, 0:1] += jnp.sum(xv, axis=-1, keepdims=True)
                acc_q[:, 0:1] += jnp.sum(xv * xv, axis=-1, keepdims=True)

            xbuf[i] = xv.astype(jnp.bfloat16)

        @pl.when(i == N - 1)
        def _finalize():
            s1 = jnp.sum(acc_s[...], axis=-1, keepdims=True)
            s2 = jnp.sum(acc_q[...], axis=-1, keepdims=True)
            mean = s1 * inv_cnt
            var = jnp.maximum(s2 * inv_cnt - mean * mean, 0.0)
            sc = g_ref[...] * jax.lax.rsqrt(var + _EPS)
            scale_r[...] = sc
            shift_r[...] = b_ref[...] - mean * sc

        @pl.when(i >= N)
        def _phase_b():
            xv = xbuf[i - N]
            sc16 = scale_r[...].astype(jnp.bfloat16)
            sh16 = shift_r[...].astype(jnp.bfloat16)
            xr = jnp.maximum(xv * sc16 + sh16, jnp.bfloat16(0.0))
            y16 = jnp.dot(wt_ref[...], xr,
                          preferred_element_type=jnp.float32
                          ).astype(jnp.bfloat16)
            d = d_ref[...]
            for gi in range(n_groups):
                z = jnp.dot(y16[:, gi * G:(gi + 1) * G], d,
                            preferred_element_type=jnp.float32)
                rows = 4 * G // (2 * W)
                slab = z.reshape(Cout, rows, 2 * W)
                row0 = gi * rows
                o_ref[:, row0:row0 + rows, :] = slab

    out = pl.pallas_call(
        body,
        out_shape=jax.ShapeDtypeStruct((N, Cout, 2 * H, 2 * W), jnp.float32),
        grid=(2 * N,),
        in_specs=[
            pl.BlockSpec((C, 1), lambda i: (0, 0)),
            pl.BlockSpec((C, 1), lambda i: (0, 0)),
            pl.BlockSpec((Cout, C), lambda i: (0, 0)),
            pl.BlockSpec((G, 4 * G), lambda i: (0, 0)),
            pl.BlockSpec((None, C, HW), lambda i: (jnp.minimum(i, N - 1), 0, 0)),
        ],
        out_specs=pl.BlockSpec((None, Cout, 2 * H, 2 * W),
                               lambda i: (jnp.maximum(i - N, 0), 0, 0, 0)),
        scratch_shapes=[
            pltpu.VMEM((N, C, HW), jnp.bfloat16),
            pltpu.VMEM((C, _LANES), jnp.float32),
            pltpu.VMEM((C, _LANES), jnp.float32),
            pltpu.VMEM((C, 1), jnp.float32),
            pltpu.VMEM((C, 1), jnp.float32),
        ],
        compiler_params=pltpu.CompilerParams(
            dimension_semantics=("arbitrary",),
            vmem_limit_bytes=_VMEM),
    )(gamma2, beta2, wt16, d16, x3)

    return out


# speedup vs baseline: 4.8184x; 1.0014x over previous
import jax
import jax.numpy as jnp
import numpy as np
from jax.experimental import pallas as pl
from jax.experimental.pallas import tpu as pltpu

_EPS = 1e-5
_LANES = 128
_VMEM = 56 * 1024 * 1024


def _width_double_matrix(G, W):
    p = np.arange(G)
    base = (4 * (p // W) * W + 2 * (p % W))[:, None]
    q = np.arange(4 * G)[None, :]
    hit = (q == base) | (q == base + 1) | (q == base + 2 * W) | (q == base + 2 * W + 1)
    return hit.astype(np.float32)


def kernel(x, w, gamma, beta):
    N, C, H, W = x.shape
    Cout = w.shape[1]
    HW = H * W
    inv_cnt = 1.0 / float(N * HW)

    G = W
    while G < 4 * W and G < 4 * _LANES and HW % (2 * G) == 0:
        G *= 2
    n_groups = HW // G
    slab_rows = 2 * G // W
    lane_chunks = HW // _LANES if HW % _LANES == 0 else 0

    x3 = x.reshape(N, C, HW)
    gamma2 = gamma.reshape(C, 1).astype(jnp.float32)
    beta2 = beta.reshape(C, 1).astype(jnp.float32)
    wt16 = w.T.astype(jnp.bfloat16)
    d16 = jnp.asarray(_width_double_matrix(G, W), dtype=jnp.bfloat16)

    def body(g_ref, b_ref, wt_ref, d_ref, x_ref, o_ref,
             xbuf, acc_s, acc_q, scale_r, shift_r):
        i = pl.program_id(0)

        @pl.when(i < N)
        def _phase_a():
            xv = x_ref[...]

            @pl.when(i == 0)
            def _init():
                acc_s[...] = jnp.zeros_like(acc_s)
                acc_q[...] = jnp.zeros_like(acc_q)

            if lane_chunks:
                s = xv[:, 0:_LANES]
                q = s * s
                for k in range(1, lane_chunks):
                    c = xv[:, k * _LANES:(k + 1) * _LANES]
                    s = s + c
                    q = q + c * c
                acc_s[...] += s
                acc_q[...] += q
            else:
                acc_s[:, 0:1] += jnp.sum(xv, axis=-1, keepdims=True)
                acc_q[:, 0:1] += jnp.sum(xv * xv, axis=-1, keepdims=True)

            xbuf[i] = xv.astype(jnp.bfloat16)

        @pl.when(i == N - 1)
        def _finalize():
            s1 = jnp.sum(acc_s[...], axis=-1, keepdims=True)
            s2 = jnp.sum(acc_q[...], axis=-1, keepdims=True)
            mean = s1 * inv_cnt
            var = jnp.maximum(s2 * inv_cnt - mean * mean, 0.0)
            sc = g_ref[...] * jax.lax.rsqrt(var + _EPS)
            scale_r[...] = sc
            shift_r[...] = b_ref[...] - mean * sc

        @pl.when(i >= N)
        def _phase_b():
            xv = xbuf[i - N]
            sc16 = scale_r[...].astype(jnp.bfloat16)
            sh16 = shift_r[...].astype(jnp.bfloat16)
            xr = jnp.maximum(xv * sc16 + sh16, jnp.bfloat16(0.0))
            y16 = jnp.dot(wt_ref[...], xr,
                          preferred_element_type=jnp.float32
                          ).astype(jnp.bfloat16)
            d = d_ref[...]
            for gi in range(n_groups):
                z = jnp.dot(y16[:, gi * G:(gi + 1) * G], d,
                            preferred_element_type=jnp.float32)
                slab = z.reshape(Cout, slab_rows, 2 * W)
                o_ref[:, gi * slab_rows:(gi + 1) * slab_rows, :] = slab

    out = pl.pallas_call(
        body,
        out_shape=jax.ShapeDtypeStruct((N, Cout, 2 * H, 2 * W), jnp.float32),
        grid=(2 * N,),
        in_specs=[
            pl.BlockSpec((C, 1), lambda i: (0, 0)),
            pl.BlockSpec((C, 1), lambda i: (0, 0)),
            pl.BlockSpec((Cout, C), lambda i: (0, 0)),
            pl.BlockSpec((G, 4 * G), lambda i: (0, 0)),
            pl.BlockSpec((None, C, HW), lambda i: (jnp.minimum(i, N - 1), 0, 0)),
        ],
        out_specs=pl.BlockSpec((None, Cout, 2 * H, 2 * W),
                               lambda i: (jnp.maximum(i - N, 0), 0, 0, 0)),
        scratch_shapes=[
            pltpu.VMEM((N, C, HW), jnp.bfloat16),
            pltpu.VMEM((C, _LANES), jnp.float32),
            pltpu.VMEM((C, _LANES), jnp.float32),
            pltpu.VMEM((C, 1), jnp.float32),
            pltpu.VMEM((C, 1), jnp.float32),
        ],
        compiler_params=pltpu.CompilerParams(
            dimension_semantics=("arbitrary",),
            vmem_limit_bytes=_VMEM),
    )(gamma2, beta2, wt16, d16, x3)

    return out
